# Initial kernel scaffold; baseline (speedup 1.0000x reference)
import functools

import jax
import jax.numpy as jnp
from jax import lax
from jax.experimental import pallas as pl
from jax.experimental.pallas import tpu as pltpu
from jax.experimental.pallas import tpu_sc as plsc

B, L, D = 4096, 200, 128
NC, NS = 2, 16
NW = NC * NS
ROWS_PER_W = B // NW
CHUNK = 100
CHUNKS_PER_ROW = L // CHUNK
CHUNKS_PER_W = ROWS_PER_W * CHUNKS_PER_ROW
NLANE = 16
NVEC = D // NLANE

_MESH = plsc.VectorSubcoreMesh(
    core_axis_name="c", subcore_axis_name="s", num_cores=NC, num_subcores=NS
)


def _rsqrt16(sv):
    i = plsc.bitcast(sv, jnp.int32)
    y = plsc.bitcast(jnp.int32(0x5F3759DF) - (i >> 1), jnp.float32)
    for _ in range(3):
        y = y * (1.5 - 0.5 * sv * y * y)
    return y


@functools.partial(
    pl.kernel,
    out_type=jax.ShapeDtypeStruct((B, D), jnp.float32),
    mesh=_MESH,
    scratch_types=[
        pltpu.VMEM((CHUNKS_PER_W, CHUNK), jnp.int32),
        pltpu.VMEM((CHUNK, D), jnp.float32),
        pltpu.VMEM((ROWS_PER_W, D), jnp.float32),
        pltpu.SemaphoreType.DMA,
    ],
)
def _sc_embed_pool(table_hbm, idx_hbm, out_hbm, idx_v, rows_v, out_v, sem):
    wid = lax.axis_index("s") * NC + lax.axis_index("c")
    cbase = wid * CHUNKS_PER_W
    pltpu.sync_copy(idx_hbm.at[pl.ds(cbase, CHUNKS_PER_W)], idx_v)

    def row_fn(r, carry):
        def chunk_fn(h, accs):
            c = r * CHUNKS_PER_ROW + h
            pltpu.async_copy(table_hbm.at[idx_v.at[c]], rows_v, sem).wait()

            def j_fn(j, a):
                return tuple(
                    a[d] + rows_v[j, pl.ds(NLANE * d, NLANE)] for d in range(NVEC)
                )

            return lax.fori_loop(0, CHUNK, j_fn, accs)

        accs = tuple(jnp.zeros((NLANE,), jnp.float32) for _ in range(NVEC))
        accs = lax.fori_loop(0, CHUNKS_PER_ROW, chunk_fn, accs)

        ssq = accs[0] * accs[0]
        for d in range(1, NVEC):
            ssq = ssq + accs[d] * accs[d]
        s = jnp.sum(ssq) * jnp.float32((1.0 / L) ** 2)
        sv = jnp.maximum(jnp.broadcast_to(s, (NLANE,)), jnp.float32(1e-12))
        scale = _rsqrt16(sv) * jnp.float32(1.0 / L)
        for d in range(NVEC):
            out_v[r, pl.ds(NLANE * d, NLANE)] = accs[d] * scale
        return carry

    lax.fori_loop(0, ROWS_PER_W, row_fn, 0)
    pltpu.sync_copy(out_v, out_hbm.at[pl.ds(wid * ROWS_PER_W, ROWS_PER_W)])


def kernel(indices, emb_table):
    idx2 = indices.reshape(B * CHUNKS_PER_ROW, CHUNK).astype(jnp.int32)
    return _sc_embed_pool(emb_table, idx2)


# speedup vs baseline: 6.1849x; 6.1849x over previous
import functools

import jax
import jax.numpy as jnp
from jax import lax
from jax.experimental import pallas as pl
from jax.experimental.pallas import tpu as pltpu
from jax.experimental.pallas import tpu_sc as plsc

B, L, D = 4096, 200, 128
NC, NS = 2, 16
NW = NC * NS
ROWS_PER_W = B // NW
CHUNK = 100
CHUNKS_PER_ROW = L // CHUNK
CHUNKS_PER_W = ROWS_PER_W * CHUNKS_PER_ROW
NLANE = 16
NVEC = D // NLANE

_MESH = plsc.VectorSubcoreMesh(
    core_axis_name="c", subcore_axis_name="s", num_cores=NC, num_subcores=NS
)


def _rsqrt16(sv):
    i = plsc.bitcast(sv, jnp.int32)
    y = plsc.bitcast(jnp.int32(0x5F3759DF) - (i >> 1), jnp.float32)
    for _ in range(3):
        y = y * (1.5 - 0.5 * sv * y * y)
    return y


@functools.partial(
    pl.kernel,
    out_type=jax.ShapeDtypeStruct((B, D), jnp.float32),
    mesh=_MESH,
    scratch_types=[
        pltpu.VMEM((CHUNKS_PER_W, CHUNK), jnp.int32),
        pltpu.VMEM((CHUNK, D), jnp.float32),
        pltpu.VMEM((ROWS_PER_W, D), jnp.float32),
        pltpu.VMEM((NLANE,), jnp.float32),
        pltpu.SemaphoreType.DMA,
    ],
    compiler_params=pltpu.CompilerParams(needs_layout_passes=False),
)
def _sc_embed_pool(table_hbm, idx_hbm, out_hbm, idx_v, rows_v, out_v, ssq_v, sem):
    wid = lax.axis_index("s") * NC + lax.axis_index("c")
    cbase = wid * CHUNKS_PER_W
    pltpu.sync_copy(idx_hbm.at[pl.ds(cbase, CHUNKS_PER_W)], idx_v)

    def row_fn(r, carry):
        def chunk_fn(h, accs):
            c = r * CHUNKS_PER_ROW + h
            pltpu.async_copy(table_hbm.at[idx_v.at[c]], rows_v, sem).wait()

            def j_fn(j, a):
                return tuple(
                    a[d] + rows_v[j, pl.ds(NLANE * d, NLANE)] for d in range(NVEC)
                )

            return lax.fori_loop(0, CHUNK, j_fn, accs)

        accs = tuple(jnp.zeros((NLANE,), jnp.float32) for _ in range(NVEC))
        accs = lax.fori_loop(0, CHUNKS_PER_ROW, chunk_fn, accs)

        ssq = accs[0] * accs[0]
        for d in range(1, NVEC):
            ssq = ssq + accs[d] * accs[d]
        s = ssq[0]
        for lane in range(1, NLANE):
            s = s + ssq[lane]
        s = s * jnp.float32((1.0 / L) ** 2)
        sv = jnp.maximum(jnp.broadcast_to(s, (NLANE,)), jnp.float32(1e-12))
        scale = _rsqrt16(sv) * jnp.float32(1.0 / L)
        for d in range(NVEC):
            out_v[r, pl.ds(NLANE * d, NLANE)] = accs[d] * scale
        return carry

    lax.fori_loop(0, ROWS_PER_W, row_fn, 0)
    pltpu.sync_copy(out_v, out_hbm.at[pl.ds(wid * ROWS_PER_W, ROWS_PER_W)])


def kernel(indices, emb_table):
    idx2 = indices.reshape(B * CHUNKS_PER_ROW, CHUNK).astype(jnp.int32)
    return _sc_embed_pool(emb_table, idx2)


# speedup vs baseline: 16.8113x; 2.7181x over previous
import functools

import jax
import jax.numpy as jnp
from jax import lax
from jax.experimental import pallas as pl
from jax.experimental.pallas import tpu as pltpu
from jax.experimental.pallas import tpu_sc as plsc

B, L, D = 4096, 200, 128
NC, NS = 2, 16
NW = NC * NS
ROWS_PER_W = B // NW
CHUNK = 100
CHUNKS_PER_ROW = L // CHUNK
CHUNKS_PER_W = ROWS_PER_W * CHUNKS_PER_ROW
NLANE = 16
NVEC = D // NLANE

_MESH = plsc.VectorSubcoreMesh(
    core_axis_name="c", subcore_axis_name="s", num_cores=NC, num_subcores=NS
)


def _rsqrt16(sv):
    i = plsc.bitcast(sv, jnp.int32)
    y = plsc.bitcast(jnp.int32(0x5F3759DF) - (i >> 1), jnp.float32)
    for _ in range(3):
        y = y * (1.5 - 0.5 * sv * y * y)
    return y


NBUF = 4
GROUPS = CHUNKS_PER_W // NBUF


@functools.partial(
    pl.kernel,
    out_type=jax.ShapeDtypeStruct((B, D), jnp.float32),
    mesh=_MESH,
    scratch_types=[
        pltpu.VMEM((CHUNKS_PER_W, CHUNK), jnp.int32),
        pltpu.VMEM((CHUNK, D), jnp.float32),
        pltpu.VMEM((CHUNK, D), jnp.float32),
        pltpu.VMEM((CHUNK, D), jnp.float32),
        pltpu.VMEM((CHUNK, D), jnp.float32),
        pltpu.VMEM((ROWS_PER_W, D), jnp.float32),
        pltpu.SemaphoreType.DMA,
        pltpu.SemaphoreType.DMA,
        pltpu.SemaphoreType.DMA,
        pltpu.SemaphoreType.DMA,
    ],
    compiler_params=pltpu.CompilerParams(needs_layout_passes=False),
)
def _sc_embed_pool(
    table_hbm, idx_hbm, out_hbm, idx_v, rows0, rows1, rows2, rows3, out_v,
    sem0, sem1, sem2, sem3
):
    sems = (sem0, sem1, sem2, sem3)
    rows = (rows0, rows1, rows2, rows3)
    wid = lax.axis_index("s") * NC + lax.axis_index("c")
    cbase = wid * CHUNKS_PER_W
    pltpu.sync_copy(idx_hbm.at[pl.ds(cbase, CHUNKS_PER_W)], idx_v)

    for b in range(NBUF):
        pltpu.async_copy(table_hbm.at[idx_v.at[b]], rows[b], sems[b])

    def group_fn(g, carry):
        for rr in range(2):
            accs = tuple(jnp.zeros((NLANE,), jnp.float32) for _ in range(NVEC))
            for h in range(CHUNKS_PER_ROW):
                k = CHUNKS_PER_ROW * rr + h

                pltpu.make_async_copy(
                    table_hbm.at[idx_v.at[0]], rows[k], sems[k]
                ).wait()

                def j_fn(j, a, _rv=rows[k]):
                    return tuple(
                        a[d] + _rv[j, pl.ds(NLANE * d, NLANE)]
                        for d in range(NVEC)
                    )

                accs = lax.fori_loop(0, CHUNK, j_fn, accs, unroll=4)

                nxt = NBUF * g + k + NBUF

                @pl.when(nxt < CHUNKS_PER_W)
                def _refill(_k=k, _nxt=nxt):
                    pltpu.async_copy(
                        table_hbm.at[idx_v.at[_nxt]], rows[_k], sems[_k]
                    )

            ssq = accs[0] * accs[0]
            for d in range(1, NVEC):
                ssq = ssq + accs[d] * accs[d]
            s = ssq[0]
            for lane in range(1, NLANE):
                s = s + ssq[lane]
            s = s * jnp.float32((1.0 / L) ** 2)
            sv = jnp.maximum(jnp.broadcast_to(s, (NLANE,)), jnp.float32(1e-12))
            scale = _rsqrt16(sv) * jnp.float32(1.0 / L)
            r = 2 * g + rr
            for d in range(NVEC):
                out_v[r, pl.ds(NLANE * d, NLANE)] = accs[d] * scale
        return carry

    lax.fori_loop(0, GROUPS, group_fn, 0)
    pltpu.sync_copy(out_v, out_hbm.at[pl.ds(wid * ROWS_PER_W, ROWS_PER_W)])


def kernel(indices, emb_table):
    idx2 = indices.reshape(B * CHUNKS_PER_ROW, CHUNK).astype(jnp.int32)
    return _sc_embed_pool(emb_table, idx2)
